# baseline (speedup 1.0000x reference)
import functools

import jax
import jax.numpy as jnp
from jax import lax
from jax.experimental import pallas as pl
from jax.experimental.pallas import tpu as pltpu
from jax.experimental.pallas import tpu_sc as plsc

_B, _L, _V = 32, 8, 100000
_R = _B * _L

_SCB = 4
_RS = _SCB * _L
_RT = _R - _RS
_B0 = _B - _SCB
_VT = (_V // 128) * 128
_TAIL = _V - _VT
_NSH = 8
_SHW = (4992, 4992, 3072)
_SH0 = 12416
_UNROLL = 4

_NOISE_CACHE = []


def _noise():
    if not _NOISE_CACHE:
        with jax.ensure_compile_time_eval():
            n = jnp.clip(
                jax.random.exponential(
                    jax.random.key(42), (_B, _L, _V), dtype=jnp.float32
                ),
                1e-10,
                None,
            )
        _NOISE_CACHE.append(n)
    return _NOISE_CACHE[0]


_ROWS = 8


def _tc_body(x_ref, t_ref, n_ref, o_ref):
    x = x_ref[...]
    t = t_ref[...]
    s = x / t
    m = jnp.max(s, axis=1, keepdims=True)
    u = jnp.exp(s - m) / n_ref[...]
    cmax = jnp.max(u, axis=1, keepdims=True)
    cols = lax.broadcasted_iota(jnp.int32, u.shape, 1)
    o_ref[...] = jnp.min(jnp.where(u == cmax, cols, _V), axis=1, keepdims=True)


def _tc_main(x, t, nrows):
    return pl.pallas_call(
        _tc_body,
        grid=(nrows // _ROWS,),
        in_specs=[
            pl.BlockSpec((_ROWS, _V), lambda i: (i, 0)),
            pl.BlockSpec((_ROWS, 1), lambda i: (i, 0)),
            pl.BlockSpec((_ROWS, _V), lambda i: (i, 0)),
        ],
        out_specs=pl.BlockSpec((_ROWS, 1), lambda i: (i, 0)),
        out_shape=jax.ShapeDtypeStruct((nrows, 1), jnp.int32),
    )(x[:nrows], t[:nrows], _noise().reshape(_R, _V)[:nrows])


def _tc_prefix_body(x_ref, t_ref, nt_ref, m_ref, tv_ref, ti_ref):
    x = x_ref[...]
    t = t_ref[...]
    m = jnp.max(x, axis=1, keepdims=True) / t
    xt = x[:, _VT:]
    u = jnp.exp(xt / t - m) / nt_ref[...]
    cmax = jnp.max(u, axis=1, keepdims=True)
    cols = lax.broadcasted_iota(jnp.int32, u.shape, 1)
    m_ref[...] = m
    tv_ref[...] = cmax
    ti_ref[...] = _VT + jnp.min(jnp.where(u == cmax, cols, _TAIL), axis=1,
                                keepdims=True)


def _tc_prefix(x, t):
    n_tail = _noise().reshape(_R, _V)[_RT:, _VT:]
    return pl.pallas_call(
        _tc_prefix_body,
        grid=(_RS // _ROWS,),
        in_specs=[
            pl.BlockSpec((_ROWS, _V), lambda i: (i, 0)),
            pl.BlockSpec((_ROWS, 1), lambda i: (i, 0)),
            pl.BlockSpec((_ROWS, _TAIL), lambda i: (i, 0)),
        ],
        out_specs=[
            pl.BlockSpec((_ROWS, 1), lambda i: (i, 0)),
            pl.BlockSpec((_ROWS, 1), lambda i: (i, 0)),
            pl.BlockSpec((_ROWS, 1), lambda i: (i, 0)),
        ],
        out_shape=[
            jax.ShapeDtypeStruct((_RS, 1), jnp.float32),
            jax.ShapeDtypeStruct((_RS, 1), jnp.float32),
            jax.ShapeDtypeStruct((_RS, 1), jnp.int32),
        ],
    )(x[_RT:], t[_RT:], n_tail)


@functools.partial(
    pl.kernel,
    mesh=plsc.VectorSubcoreMesh(core_axis_name="c", subcore_axis_name="s"),
    out_type=(
        jax.ShapeDtypeStruct((32 * 16,), jnp.float32),
        jax.ShapeDtypeStruct((32 * 16,), jnp.int32),
    ),
    scratch_types=[
        pltpu.VMEM((_ROWS, 4992), jnp.float32),
        pltpu.VMEM((_ROWS, 4992), jnp.float32),
        pltpu.VMEM((16,), jnp.float32),
        pltpu.VMEM((16,), jnp.float32),
        pltpu.VMEM((16,), jnp.float32),
        pltpu.VMEM((16,), jnp.int32),
    ],
)
def _sc_part(x_hbm, t_hbm, m_hbm, n_hbm, vout_hbm, iout_hbm,
             xbuf, nbuf, tvec, mvec, resv, resi):
    wid = lax.axis_index("s") * 2 + lax.axis_index("c")
    brel = wid // _NSH
    q = wid % _NSH
    b = _B0 + brel
    c0 = q * _SH0
    iota = lax.iota(jnp.int32, 16)
    pltpu.sync_copy(t_hbm.at[pl.ds(16 * b, 16)], tvec)
    t_v = tvec[...]
    m_list = []
    for r in range(_L):
        pltpu.sync_copy(m_hbm.at[pl.ds(16 * (brel * _L + r), 16)], mvec)
        m_list.append(mvec[...])
    vmax = [jnp.full((16,), -jnp.inf, jnp.float32) for _ in range(_L)]
    vidx = [jnp.zeros((16,), jnp.int32) for _ in range(_L)]
    off = 0
    for cw in _SHW:
        pltpu.sync_copy(x_hbm.at[b, :, pl.ds(c0 + off, cw)],
                        xbuf.at[:, pl.ds(0, cw)])
        pltpu.sync_copy(n_hbm.at[b, :, pl.ds(c0 + off, cw)],
                        nbuf.at[:, pl.ds(0, cw)])
        nvec = cw // (16 * _UNROLL)
        base = c0 + off
        for r in range(_L):
            t_r = t_v
            m_r = m_list[r]

            def bstep(i, carry, r=r, t_r=t_r, m_r=m_r):
                bm, bi = carry
                for u in range(_UNROLL):
                    o = (i * _UNROLL + u) * 16
                    x_v = xbuf[r, pl.ds(o, 16)]
                    n_v = nbuf[r, pl.ds(o, 16)]
                    uu = jnp.exp(x_v / t_r - m_r) / n_v
                    take = uu > bm
                    bm = jnp.maximum(bm, uu)
                    bi = jnp.where(take, iota + (base + o), bi)
                return (bm, bi)

            vmax[r], vidx[r] = lax.fori_loop(
                0, nvec, bstep, (vmax[r], vidx[r]))
        off += cw
    def shuf(v, d):
        return lax.gather(
            v,
            jnp.bitwise_xor(iota, d)[:, None],
            lax.GatherDimensionNumbers(
                offset_dims=(), collapsed_slice_dims=(0,),
                start_index_map=(0,)),
            slice_sizes=(1,),
            mode=lax.GatherScatterMode.PROMISE_IN_BOUNDS,
        )

    res_val = jnp.full((16,), -jnp.inf, jnp.float32)
    res_idx = jnp.zeros((16,), jnp.int32)
    for r in range(_L):
        rmax = vmax[r]
        for d in (8, 4, 2, 1):
            rmax = jnp.maximum(rmax, shuf(rmax, d))
        cand = jnp.where(vmax[r] == rmax, vidx[r], _V)
        for d in (8, 4, 2, 1):
            cand = jnp.minimum(cand, shuf(cand, d))
        res_val = jnp.where(iota == r, rmax, res_val)
        res_idx = jnp.where(iota == r, cand, res_idx)
    resv[...] = res_val
    resi[...] = res_idx
    pltpu.sync_copy(resv, vout_hbm.at[pl.ds(16 * wid, 16)])
    pltpu.sync_copy(resi, iout_hbm.at[pl.ds(16 * wid, 16)])


def kernel(logits, temperatures):
    B, L, V = logits.shape
    x = logits.reshape(B * L, V)
    tt = temperatures.astype(jnp.float32)
    t = jnp.broadcast_to(tt[:, None], (B, L)).reshape(B * L, 1)
    m_sc, tail_val, tail_idx = _tc_prefix(x, t)
    t512 = jnp.broadcast_to(tt[:, None], (B, 16)).reshape(B * 16)
    msc16 = jnp.broadcast_to(m_sc, (_RS, 16)).reshape(_RS * 16)
    sc_val, sc_idx = _sc_part(logits, t512, msc16, _noise())
    out_tc = _tc_main(x, t, _RT)
    v4 = sc_val.reshape(_SCB, _NSH, 16)[:, :, :_L]
    i4 = sc_idx.reshape(_SCB, _NSH, 16)[:, :, :_L]
    vbest = jnp.max(v4, axis=1)
    ibest = jnp.min(jnp.where(v4 == vbest[:, None, :], i4, _V), axis=1)
    tv = tail_val.reshape(_SCB, _L)
    ti = tail_idx.reshape(_SCB, _L)
    sc_rows = jnp.where(vbest >= tv, ibest, ti).reshape(_RS)
    out = jnp.concatenate([out_tc.reshape(_RT), sc_rows])
    return out.reshape(B, L)


# speedup vs baseline: 2.6268x; 2.6268x over previous
import jax
import jax.numpy as jnp
from jax import lax
from jax.experimental import pallas as pl

_B, _L, _V = 32, 8, 100000
_R = _B * _L

_NOISE_CACHE = []


def _noise():
    if not _NOISE_CACHE:
        with jax.ensure_compile_time_eval():
            n = jnp.clip(
                jax.random.exponential(
                    jax.random.key(42), (_B, _L, _V), dtype=jnp.float32
                ),
                1e-10,
                None,
            ).reshape(_R, _V)
        _NOISE_CACHE.append(n)
    return _NOISE_CACHE[0]


_ROWS = 8


def _body(x_ref, t_ref, n_ref, o_ref):
    x = x_ref[...]
    t = t_ref[...]
    s = x / t
    m = jnp.max(s, axis=1, keepdims=True)
    u = jnp.exp(s - m) / n_ref[...]
    cmax = jnp.max(u, axis=1, keepdims=True)
    cols = lax.broadcasted_iota(jnp.int32, u.shape, 1)
    o_ref[...] = jnp.min(jnp.where(u == cmax, cols, _V), axis=1, keepdims=True)


def kernel(logits, temperatures):
    B, L, V = logits.shape
    x = logits.reshape(B * L, V)
    t = jnp.broadcast_to(temperatures.astype(jnp.float32)[:, None], (B, L)).reshape(
        B * L, 1
    )
    out = pl.pallas_call(
        _body,
        grid=(B * L // _ROWS,),
        in_specs=[
            pl.BlockSpec((_ROWS, V), lambda i: (i, 0)),
            pl.BlockSpec((_ROWS, 1), lambda i: (i, 0)),
            pl.BlockSpec((_ROWS, V), lambda i: (i, 0)),
        ],
        out_specs=pl.BlockSpec((_ROWS, 1), lambda i: (i, 0)),
        out_shape=jax.ShapeDtypeStruct((B * L, 1), jnp.int32),
    )(x, t, _noise())
    return out.reshape(B, L)
